# baseline (speedup 1.0000x reference)
import functools

import numpy as np

import jax
import jax.numpy as jnp
from jax import lax
from jax.experimental import pallas as pl
from jax.experimental.pallas import tpu as pltpu
from jax.experimental.pallas import tpu_sc as plsc

D = 128
EMBED = 128
S = 10
NW = 32
BPW = 1568
B_PAD = NW * BPW
TILE = 8
NT = BPW // TILE
RING = 4
C = 2 * D

_mesh = plsc.VectorSubcoreMesh(core_axis_name="c", subcore_axis_name="s")


def _tree_sum(vals):
    while len(vals) > 1:
        nxt = [vals[i] + vals[i + 1] for i in range(0, len(vals) - 1, 2)]
        if len(vals) % 2:
            nxt.append(vals[-1])
        vals = nxt
    return vals[0]


def _pack_bf16_words(a, b):
    ai = lax.bitcast_convert_type(a, jnp.int32)
    bi = lax.bitcast_convert_type(b, jnp.int32)
    ar = ai + jnp.int32(0x7FFF) + ((ai >> 16) & jnp.int32(1))
    br = bi + jnp.int32(0x7FFF) + ((bi >> 16) & jnp.int32(1))
    lo = (ar >> 16) & jnp.int32(0xFFFF)
    hi = br & jnp.int32(-0x10000)
    return hi | lo


CW = C // 2


def _accum(selfb, rows, stage):
    def node(t, c):
        r0 = t * S
        o0 = t * CW
        for p in range(D // 32):
            lo = pl.ds(32 * p, 16)
            hi = pl.ds(32 * p + 16, 16)
            stage[pl.ds(o0 + 16 * p, 16)] = _pack_bf16_words(
                selfb[t, lo], selfb[t, hi])
            sa = _tree_sum([rows[r0 + j, lo] for j in range(S)])
            sb = _tree_sum([rows[r0 + j, hi] for j in range(S)])
            stage[pl.ds(o0 + D // 2 + 16 * p, 16)] = _pack_bf16_words(sa, sb)
        return c

    lax.fori_loop(0, TILE, node, 0)


@functools.partial(
    pl.kernel,
    out_type=jax.ShapeDtypeStruct((B_PAD * CW,), jnp.int32),
    mesh=_mesh,
    scratch_types=(
        (pltpu.VMEM((NT, TILE), jnp.int32),
         pltpu.VMEM((NT, TILE * S), jnp.int32))
        + tuple(pltpu.VMEM((TILE, D), jnp.float32)
                for _ in range(RING))
        + tuple(pltpu.VMEM((TILE * S, D), jnp.float32)
                for _ in range(RING))
        + tuple(pltpu.VMEM((TILE * CW,), jnp.int32)
                for _ in range(RING))
        + tuple(pltpu.SemaphoreType.DMA for _ in range(3 * RING))
    ),
)
def _sc_gather(nodes_hbm, neigh_hbm, feat_hbm, comb_out,
               idx_sv, idx_nv, *bufs):
    wid = lax.axis_index("s") * 2 + lax.axis_index("c")
    base = wid * BPW
    selfbs = bufs[:RING]
    rows = bufs[RING:2 * RING]
    stages = bufs[2 * RING:3 * RING]
    ssem = bufs[3 * RING:4 * RING]
    gsem = bufs[4 * RING:5 * RING]
    wsem = bufs[5 * RING:6 * RING]

    pltpu.sync_copy(nodes_hbm.at[wid], idx_sv)
    pltpu.sync_copy(neigh_hbm.at[wid], idx_nv)
    for b in range(RING):
        pltpu.async_copy(feat_hbm.at[idx_sv.at[b]], selfbs[b], ssem[b])
        pltpu.async_copy(feat_hbm.at[idx_nv.at[b]], rows[b], gsem[b])

    def body(k, c):
        for b in range(RING):
            i = k * RING + b
            st = stages[b]
            pltpu.make_async_copy(feat_hbm.at[idx_sv.at[i]], selfbs[b],
                                  ssem[b]).wait()
            pltpu.make_async_copy(feat_hbm.at[idx_nv.at[i]], rows[b],
                                  gsem[b]).wait()

            def _wait_prev_write():
                pltpu.make_async_copy(
                    st,
                    comb_out.at[pl.ds((base + (i - RING) * TILE) * CW,
                                      TILE * CW)],
                    wsem[b]).wait()

            pl.when(k > 0)(_wait_prev_write)

            _accum(selfbs[b], rows[b], st)
            pltpu.async_copy(
                st, comb_out.at[pl.ds((base + i * TILE) * CW, TILE * CW)],
                wsem[b])

            def _next_gather():
                pltpu.async_copy(feat_hbm.at[idx_sv.at[i + RING]],
                                 selfbs[b], ssem[b])
                pltpu.async_copy(feat_hbm.at[idx_nv.at[i + RING]],
                                 rows[b], gsem[b])

            pl.when(k < NT // RING - 1)(_next_gather)
        return c

    lax.fori_loop(0, NT // RING, body, 0)
    for b in range(RING):
        pltpu.make_async_copy(
            stages[b],
            comb_out.at[pl.ds((base + (NT - RING + b) * TILE) * CW,
                              TILE * CW)],
            wsem[b]).wait()


_PERM = np.empty((C,), dtype=np.int32)
for _c in range(C):
    _g, _r = _c // 32, _c % 32
    _PERM[_c] = 32 * _g + _r // 2 + 16 * (_r % 2)

BLK = 512
_DN = (((1,), (1,)), ((), ()))


def _tc_body(w_ref, comb_ref, out_ref):
    out = lax.dot_general(w_ref[...].astype(jnp.bfloat16), comb_ref[...],
                          _DN, preferred_element_type=jnp.float32)
    out_ref[...] = jnp.maximum(out, 0.0)


def _tc_call(b):
    return pl.pallas_call(
        _tc_body,
        grid=(B_PAD // BLK,),
        in_specs=[
            pl.BlockSpec((EMBED, C), lambda i: (0, 0)),
            pl.BlockSpec((BLK, C), lambda i: (i, 0)),
        ],
        out_specs=pl.BlockSpec((EMBED, BLK), lambda i: (0, i)),
        out_shape=jax.ShapeDtypeStruct((EMBED, b), jnp.float32),
    )


def kernel(nodes, neigh_idx, features, weight, pos_index, neg_index):
    del pos_index, neg_index
    b = nodes.shape[0]
    nodes_p = jnp.pad(nodes.astype(jnp.int32), (0, B_PAD - b))
    neigh_p = jnp.pad(neigh_idx.astype(jnp.int32),
                      ((0, B_PAD - b), (0, 0)))
    nodes3 = nodes_p.reshape(NW, NT, TILE)
    neigh3 = neigh_p.reshape(NW, NT, TILE * S)
    comb_words = _sc_gather(nodes3, neigh3, features)
    comb = lax.bitcast_convert_type(
        comb_words, jnp.bfloat16).reshape(B_PAD, C)
    w2 = jnp.concatenate(
        [weight[:, :D], weight[:, D:] * jnp.float32(1.0 / S)], axis=1)
    w2 = w2[:, _PERM]
    return _tc_call(b)(w2, comb)


# speedup vs baseline: 1.7141x; 1.7141x over previous
import functools

import numpy as np

import jax
import jax.numpy as jnp
from jax import lax
from jax.experimental import pallas as pl
from jax.experimental.pallas import tpu as pltpu
from jax.experimental.pallas import tpu_sc as plsc

D = 128
EMBED = 128
S = 10
NW = 32
BPW = 1568
B_PAD = NW * BPW
TILE = 8
NT = BPW // TILE
RING = 4
C = 2 * D
CW = C // 2

_mesh = plsc.VectorSubcoreMesh(core_axis_name="c", subcore_axis_name="s")


def _tree_sum(vals):
    while len(vals) > 1:
        nxt = [vals[i] + vals[i + 1] for i in range(0, len(vals) - 1, 2)]
        if len(vals) % 2:
            nxt.append(vals[-1])
        vals = nxt
    return vals[0]


def _pack_bf16_words(a, b):
    ai = lax.bitcast_convert_type(a, jnp.int32)
    bi = lax.bitcast_convert_type(b, jnp.int32)
    ar = ai + jnp.int32(0x7FFF) + ((ai >> 16) & jnp.int32(1))
    br = bi + jnp.int32(0x7FFF) + ((bi >> 16) & jnp.int32(1))
    lo = (ar >> 16) & jnp.int32(0xFFFF)
    hi = br & jnp.int32(-0x10000)
    return hi | lo


def _accum(selfb, rows, stage):
    def node(t, c):
        r0 = t * S
        for p in range(D // 32):
            lo = pl.ds(32 * p, 16)
            hi = pl.ds(32 * p + 16, 16)
            stage[t, pl.ds(16 * p, 16)] = _pack_bf16_words(
                selfb[t, lo], selfb[t, hi])
            sa = _tree_sum([rows[r0 + j, lo] for j in range(S)])
            sb = _tree_sum([rows[r0 + j, hi] for j in range(S)])
            stage[t, pl.ds(D // 2 + 16 * p, 16)] = _pack_bf16_words(sa, sb)
        return c

    lax.fori_loop(0, TILE, node, 0)


@functools.partial(
    pl.kernel,
    out_type=jax.ShapeDtypeStruct((B_PAD, CW), jnp.int32),
    mesh=_mesh,
    scratch_types=(
        (pltpu.VMEM((NT, TILE), jnp.int32),
         pltpu.VMEM((NT, TILE * S), jnp.int32))
        + tuple(pltpu.VMEM((TILE, D), jnp.float32)
                for _ in range(RING))
        + tuple(pltpu.VMEM((TILE * S, D), jnp.float32)
                for _ in range(RING))
        + tuple(pltpu.VMEM((TILE, CW), jnp.int32)
                for _ in range(RING))
        + tuple(pltpu.SemaphoreType.DMA for _ in range(3 * RING))
    ),
)
def _sc_gather(nodes_hbm, neigh_hbm, feat_hbm, comb_out,
               idx_sv, idx_nv, *bufs):
    wid = lax.axis_index("s") * 2 + lax.axis_index("c")
    base = wid * BPW
    selfbs = bufs[:RING]
    rows = bufs[RING:2 * RING]
    stages = bufs[2 * RING:3 * RING]
    ssem = bufs[3 * RING:4 * RING]
    gsem = bufs[4 * RING:5 * RING]
    wsem = bufs[5 * RING:6 * RING]

    pltpu.sync_copy(nodes_hbm.at[wid], idx_sv)
    pltpu.sync_copy(neigh_hbm.at[wid], idx_nv)
    for b in range(RING):
        pltpu.async_copy(feat_hbm.at[idx_sv.at[b]], selfbs[b], ssem[b])
        pltpu.async_copy(feat_hbm.at[idx_nv.at[b]], rows[b], gsem[b])

    def body(k, c):
        for b in range(RING):
            i = k * RING + b
            st = stages[b]
            pltpu.make_async_copy(feat_hbm.at[idx_sv.at[i]], selfbs[b],
                                  ssem[b]).wait()
            pltpu.make_async_copy(feat_hbm.at[idx_nv.at[i]], rows[b],
                                  gsem[b]).wait()

            def _wait_prev_write():
                pltpu.make_async_copy(
                    st,
                    comb_out.at[pl.ds(base + (i - RING) * TILE, TILE)],
                    wsem[b]).wait()

            pl.when(k > 0)(_wait_prev_write)

            _accum(selfbs[b], rows[b], st)
            pltpu.async_copy(
                st, comb_out.at[pl.ds(base + i * TILE, TILE)], wsem[b])

            def _next_gather():
                pltpu.async_copy(feat_hbm.at[idx_sv.at[i + RING]],
                                 selfbs[b], ssem[b])
                pltpu.async_copy(feat_hbm.at[idx_nv.at[i + RING]],
                                 rows[b], gsem[b])

            pl.when(k < NT // RING - 1)(_next_gather)
        return c

    lax.fori_loop(0, NT // RING, body, 0)
    for b in range(RING):
        pltpu.make_async_copy(
            stages[b],
            comb_out.at[pl.ds(base + (NT - RING + b) * TILE, TILE)],
            wsem[b]).wait()


_PERM = np.empty((C,), dtype=np.int32)
for _w in range(CW):
    _half = _w // 64
    _g = (_w % 64) // 16
    _k = _w % 16
    _PERM[_w] = 128 * _half + 32 * _g + _k
    _PERM[CW + _w] = 128 * _half + 32 * _g + 16 + _k

BLK = 1024
_DN = (((1,), (1,)), ((), ()))


def _tc_body(w_ref, comb_ref, out_ref):
    words = comb_ref[...]
    lo = lax.bitcast_convert_type(words << 16, jnp.float32)
    hi = lax.bitcast_convert_type(words & jnp.int32(-0x10000), jnp.float32)
    x = jnp.concatenate([lo, hi], axis=1).astype(jnp.bfloat16)
    out = lax.dot_general(w_ref[...].astype(jnp.bfloat16), x,
                          _DN, preferred_element_type=jnp.float32)
    out_ref[...] = jnp.maximum(out, 0.0)


def _tc_call(b):
    return pl.pallas_call(
        _tc_body,
        grid=(B_PAD // BLK,),
        in_specs=[
            pl.BlockSpec((EMBED, C), lambda i: (0, 0)),
            pl.BlockSpec((BLK, CW), lambda i: (i, 0)),
        ],
        out_specs=pl.BlockSpec((EMBED, BLK), lambda i: (0, i)),
        out_shape=jax.ShapeDtypeStruct((EMBED, b), jnp.float32),
    )


def kernel(nodes, neigh_idx, features, weight, pos_index, neg_index):
    del pos_index, neg_index
    b = nodes.shape[0]
    nodes_p = jnp.pad(nodes.astype(jnp.int32), (0, B_PAD - b))
    neigh_p = jnp.pad(neigh_idx.astype(jnp.int32),
                      ((0, B_PAD - b), (0, 0)))
    nodes3 = nodes_p.reshape(NW, NT, TILE)
    neigh3 = neigh_p.reshape(NW, NT, TILE * S)
    comb_words = _sc_gather(nodes3, neigh3, features)
    w2 = jnp.concatenate(
        [weight[:, :D], weight[:, D:] * jnp.float32(1.0 / S)], axis=1)
    w2 = w2[:, _PERM]
    return _tc_call(b)(w2, comb_words)


# speedup vs baseline: 1.7790x; 1.0379x over previous
import functools

import numpy as np

import jax
import jax.numpy as jnp
from jax import lax
from jax.experimental import pallas as pl
from jax.experimental.pallas import tpu as pltpu
from jax.experimental.pallas import tpu_sc as plsc

D = 128
EMBED = 128
S = 10
NW = 32
BPW = 1568
B_PAD = NW * BPW
TILE = 8
NT = BPW // TILE
RING = 4
C = 2 * D
CW = C // 2

_mesh = plsc.VectorSubcoreMesh(core_axis_name="c", subcore_axis_name="s")


def _tree_sum(vals):
    while len(vals) > 1:
        nxt = [vals[i] + vals[i + 1] for i in range(0, len(vals) - 1, 2)]
        if len(vals) % 2:
            nxt.append(vals[-1])
        vals = nxt
    return vals[0]


def _pack_bf16_words(a, b):
    ai = lax.bitcast_convert_type(a, jnp.int32)
    bi = lax.bitcast_convert_type(b, jnp.int32)
    ar = ai + jnp.int32(0x7FFF) + ((ai >> 16) & jnp.int32(1))
    br = bi + jnp.int32(0x7FFF) + ((bi >> 16) & jnp.int32(1))
    lo = (ar >> 16) & jnp.int32(0xFFFF)
    hi = br & jnp.int32(-0x10000)
    return hi | lo


def _accum(selfb, rows, stage):
    def node(t, c):
        r0 = t * S
        for p in range(D // 32):
            lo = pl.ds(32 * p, 16)
            hi = pl.ds(32 * p + 16, 16)
            stage[t, pl.ds(16 * p, 16)] = _pack_bf16_words(
                selfb[t, lo], selfb[t, hi])
            sa = _tree_sum([rows[r0 + j, lo] for j in range(S)])
            sb = _tree_sum([rows[r0 + j, hi] for j in range(S)])
            stage[t, pl.ds(D // 2 + 16 * p, 16)] = _pack_bf16_words(sa, sb)
        return c

    lax.fori_loop(0, TILE, node, 0)


@functools.partial(
    pl.kernel,
    out_type=jax.ShapeDtypeStruct((B_PAD, CW), jnp.int32),
    mesh=_mesh,
    scratch_types=(
        (pltpu.VMEM((BPW,), jnp.int32),
         pltpu.VMEM((BPW * S,), jnp.int32))
        + tuple(pltpu.VMEM((TILE, D), jnp.float32)
                for _ in range(RING))
        + tuple(pltpu.VMEM((TILE * S, D), jnp.float32)
                for _ in range(RING))
        + tuple(pltpu.VMEM((TILE, CW), jnp.int32)
                for _ in range(RING))
        + tuple(pltpu.SemaphoreType.DMA for _ in range(3 * RING))
    ),
)
def _sc_gather(nodes_hbm, neigh_hbm, feat_hbm, comb_out,
               idx_sv, idx_nv, *bufs):
    wid = lax.axis_index("s") * 2 + lax.axis_index("c")
    base = wid * BPW
    selfbs = bufs[:RING]
    rows = bufs[RING:2 * RING]
    stages = bufs[2 * RING:3 * RING]
    ssem = bufs[3 * RING:4 * RING]
    gsem = bufs[4 * RING:5 * RING]
    wsem = bufs[5 * RING:6 * RING]

    pltpu.sync_copy(nodes_hbm.at[pl.ds(base, BPW)], idx_sv)
    pltpu.sync_copy(neigh_hbm.at[pl.ds(base * S, BPW * S)], idx_nv)

    def _sidx(i):
        return idx_sv.at[pl.ds(i * TILE, TILE)]

    def _nidx(i):
        return idx_nv.at[pl.ds(i * TILE * S, TILE * S)]

    for b in range(RING):
        pltpu.async_copy(feat_hbm.at[_sidx(b)], selfbs[b], ssem[b])
        pltpu.async_copy(feat_hbm.at[_nidx(b)], rows[b], gsem[b])

    def body(k, c):
        for b in range(RING):
            i = k * RING + b
            st = stages[b]
            pltpu.make_async_copy(feat_hbm.at[_sidx(i)], selfbs[b],
                                  ssem[b]).wait()
            pltpu.make_async_copy(feat_hbm.at[_nidx(i)], rows[b],
                                  gsem[b]).wait()

            def _wait_prev_write():
                pltpu.make_async_copy(
                    st,
                    comb_out.at[pl.ds(base + (i - RING) * TILE, TILE)],
                    wsem[b]).wait()

            pl.when(k > 0)(_wait_prev_write)

            _accum(selfbs[b], rows[b], st)
            pltpu.async_copy(
                st, comb_out.at[pl.ds(base + i * TILE, TILE)], wsem[b])

            def _next_gather():
                pltpu.async_copy(feat_hbm.at[_sidx(i + RING)],
                                 selfbs[b], ssem[b])
                pltpu.async_copy(feat_hbm.at[_nidx(i + RING)],
                                 rows[b], gsem[b])

            pl.when(k < NT // RING - 1)(_next_gather)
        return c

    lax.fori_loop(0, NT // RING, body, 0)
    for b in range(RING):
        pltpu.make_async_copy(
            stages[b],
            comb_out.at[pl.ds(base + (NT - RING + b) * TILE, TILE)],
            wsem[b]).wait()


_PERM = np.empty((C,), dtype=np.int32)
for _w in range(CW):
    _half = _w // 64
    _g = (_w % 64) // 16
    _k = _w % 16
    _PERM[_w] = 128 * _half + 32 * _g + _k
    _PERM[CW + _w] = 128 * _half + 32 * _g + 16 + _k

BLK = 1024
_DN = (((1,), (1,)), ((), ()))


def _tc_body(w_ref, comb_ref, out_ref):
    words = comb_ref[...]
    lo = lax.bitcast_convert_type(words << 16, jnp.float32)
    hi = lax.bitcast_convert_type(words & jnp.int32(-0x10000), jnp.float32)
    x = jnp.concatenate([lo, hi], axis=1).astype(jnp.bfloat16)
    out = lax.dot_general(w_ref[...].astype(jnp.bfloat16), x,
                          _DN, preferred_element_type=jnp.float32)
    out_ref[...] = jnp.maximum(out, 0.0)


def _tc_call(b):
    return pl.pallas_call(
        _tc_body,
        grid=(B_PAD // BLK,),
        in_specs=[
            pl.BlockSpec((EMBED, C), lambda i: (0, 0)),
            pl.BlockSpec((BLK, CW), lambda i: (i, 0)),
        ],
        out_specs=pl.BlockSpec((EMBED, BLK), lambda i: (0, i)),
        out_shape=jax.ShapeDtypeStruct((EMBED, b), jnp.float32),
    )


def kernel(nodes, neigh_idx, features, weight, pos_index, neg_index):
    del pos_index, neg_index
    b = nodes.shape[0]
    nodes_p = jnp.pad(nodes.astype(jnp.int32), (0, B_PAD - b))
    neigh_p = jnp.pad(neigh_idx.astype(jnp.int32).reshape(-1),
                      (0, (B_PAD - b) * S))
    comb_words = _sc_gather(nodes_p, neigh_p, features)
    w2 = jnp.concatenate(
        [weight[:, :D], weight[:, D:] * jnp.float32(1.0 / S)], axis=1)
    w2 = w2[:, _PERM]
    return _tc_call(b)(w2, comb_words)


# speedup vs baseline: 1.8551x; 1.0428x over previous
import functools

import numpy as np

import jax
import jax.numpy as jnp
from jax import lax
from jax.experimental import pallas as pl
from jax.experimental.pallas import tpu as pltpu
from jax.experimental.pallas import tpu_sc as plsc

D = 128
EMBED = 128
S = 10
NW = 32
BPW = 1568
B_PAD = NW * BPW
TILE = 8
NT = BPW // TILE
RING = 4
NT_A = 244
NT_B = 148
PAD_EXTRA = (NT_A - NT_B) * TILE
C = 2 * D
CW = C // 2

_mesh = plsc.VectorSubcoreMesh(core_axis_name="c", subcore_axis_name="s")


def _tree_sum(vals):
    while len(vals) > 1:
        nxt = [vals[i] + vals[i + 1] for i in range(0, len(vals) - 1, 2)]
        if len(vals) % 2:
            nxt.append(vals[-1])
        vals = nxt
    return vals[0]


def _pack_bf16_words(a, b):
    ai = lax.bitcast_convert_type(a, jnp.int32)
    bi = lax.bitcast_convert_type(b, jnp.int32)
    ar = ai + jnp.int32(0x7FFF) + ((ai >> 16) & jnp.int32(1))
    br = bi + jnp.int32(0x7FFF) + ((bi >> 16) & jnp.int32(1))
    lo = (ar >> 16) & jnp.int32(0xFFFF)
    hi = br & jnp.int32(-0x10000)
    return hi | lo


def _accum(selfb, rows, stage):
    def node(t, c):
        r0 = t * S
        for p in range(D // 32):
            lo = pl.ds(32 * p, 16)
            hi = pl.ds(32 * p + 16, 16)
            stage[t, pl.ds(16 * p, 16)] = _pack_bf16_words(
                selfb[t, lo], selfb[t, hi])
            sa = _tree_sum([rows[r0 + j, lo] for j in range(S)])
            sb = _tree_sum([rows[r0 + j, hi] for j in range(S)])
            stage[t, pl.ds(D // 2 + 16 * p, 16)] = _pack_bf16_words(sa, sb)
        return c

    lax.fori_loop(0, TILE, node, 0)


@functools.partial(
    pl.kernel,
    out_type=jax.ShapeDtypeStruct((B_PAD, CW), jnp.int32),
    mesh=_mesh,
    scratch_types=(
        (pltpu.VMEM((NT_A * TILE,), jnp.int32),
         pltpu.VMEM((NT_A * TILE * S,), jnp.int32))
        + tuple(pltpu.VMEM((TILE, D), jnp.float32)
                for _ in range(RING))
        + tuple(pltpu.VMEM((TILE * S, D), jnp.float32)
                for _ in range(RING))
        + tuple(pltpu.VMEM((TILE, CW), jnp.int32)
                for _ in range(RING))
        + tuple(pltpu.SemaphoreType.DMA for _ in range(3 * RING))
    ),
)
def _sc_gather(nodes_hbm, neigh_hbm, feat_hbm, comb_out,
               idx_sv, idx_nv, *bufs):
    cid = lax.axis_index("c")
    sid = lax.axis_index("s")
    nt = lax.select(cid == 0, NT_A, NT_B)
    base_tile = lax.select(cid == 0, sid * NT_A, 16 * NT_A + sid * NT_B)
    base = base_tile * TILE
    selfbs = bufs[:RING]
    rows = bufs[RING:2 * RING]
    stages = bufs[2 * RING:3 * RING]
    ssem = bufs[3 * RING:4 * RING]
    gsem = bufs[4 * RING:5 * RING]
    wsem = bufs[5 * RING:6 * RING]

    pltpu.sync_copy(nodes_hbm.at[pl.ds(base, NT_A * TILE)], idx_sv)
    pltpu.sync_copy(neigh_hbm.at[pl.ds(base * S, NT_A * TILE * S)], idx_nv)

    def _sidx(i):
        return idx_sv.at[pl.ds(i * TILE, TILE)]

    def _nidx(i):
        return idx_nv.at[pl.ds(i * TILE * S, TILE * S)]

    for b in range(RING):
        pltpu.async_copy(feat_hbm.at[_sidx(b)], selfbs[b], ssem[b])
        pltpu.async_copy(feat_hbm.at[_nidx(b)], rows[b], gsem[b])

    def body(k, c):
        for b in range(RING):
            i = k * RING + b
            st = stages[b]
            pltpu.make_async_copy(feat_hbm.at[_sidx(i)], selfbs[b],
                                  ssem[b]).wait()
            pltpu.make_async_copy(feat_hbm.at[_nidx(i)], rows[b],
                                  gsem[b]).wait()

            def _wait_prev_write():
                pltpu.make_async_copy(
                    st,
                    comb_out.at[pl.ds(base + (i - RING) * TILE, TILE)],
                    wsem[b]).wait()

            pl.when(k > 0)(_wait_prev_write)

            _accum(selfbs[b], rows[b], st)
            pltpu.async_copy(
                st, comb_out.at[pl.ds(base + i * TILE, TILE)], wsem[b])

            def _next_gather():
                pltpu.async_copy(feat_hbm.at[_sidx(i + RING)],
                                 selfbs[b], ssem[b])
                pltpu.async_copy(feat_hbm.at[_nidx(i + RING)],
                                 rows[b], gsem[b])

            pl.when(k < nt // RING - 1)(_next_gather)
        return c

    lax.fori_loop(0, nt // RING, body, 0)
    for b in range(RING):
        pltpu.make_async_copy(
            stages[b],
            comb_out.at[pl.ds(base + (nt - RING + b) * TILE, TILE)],
            wsem[b]).wait()


_PERM = np.empty((C,), dtype=np.int32)
for _w in range(CW):
    _half = _w // 64
    _g = (_w % 64) // 16
    _k = _w % 16
    _PERM[_w] = 128 * _half + 32 * _g + _k
    _PERM[CW + _w] = 128 * _half + 32 * _g + 16 + _k

BLK = 1024
_DN = (((1,), (1,)), ((), ()))


def _tc_body(w_ref, comb_ref, out_ref):
    words = comb_ref[...]
    lo = lax.bitcast_convert_type(words << 16, jnp.float32)
    hi = lax.bitcast_convert_type(words & jnp.int32(-0x10000), jnp.float32)
    x = jnp.concatenate([lo, hi], axis=1).astype(jnp.bfloat16)
    out = lax.dot_general(w_ref[...].astype(jnp.bfloat16), x,
                          _DN, preferred_element_type=jnp.float32)
    out_ref[...] = jnp.maximum(out, 0.0)


def _tc_call(b):
    return pl.pallas_call(
        _tc_body,
        grid=(B_PAD // BLK,),
        in_specs=[
            pl.BlockSpec((EMBED, C), lambda i: (0, 0)),
            pl.BlockSpec((BLK, CW), lambda i: (i, 0)),
        ],
        out_specs=pl.BlockSpec((EMBED, BLK), lambda i: (0, i)),
        out_shape=jax.ShapeDtypeStruct((EMBED, b), jnp.float32),
    )


def kernel(nodes, neigh_idx, features, weight, pos_index, neg_index):
    del pos_index, neg_index
    b = nodes.shape[0]
    nodes_p = jnp.pad(nodes.astype(jnp.int32), (0, B_PAD - b + PAD_EXTRA))
    neigh_p = jnp.pad(neigh_idx.astype(jnp.int32).reshape(-1),
                      (0, (B_PAD - b + PAD_EXTRA) * S))
    comb_words = _sc_gather(nodes_p, neigh_p, features)
    w2 = jnp.concatenate(
        [weight[:, :D], weight[:, D:] * jnp.float32(1.0 / S)], axis=1)
    w2 = w2[:, _PERM]
    return _tc_call(b)(w2, comb_words)


# speedup vs baseline: 1.9064x; 1.0277x over previous
import functools

import numpy as np

import jax
import jax.numpy as jnp
from jax import lax
from jax.experimental import pallas as pl
from jax.experimental.pallas import tpu as pltpu
from jax.experimental.pallas import tpu_sc as plsc

D = 128
EMBED = 128
S = 10
NW = 32
BPW = 1568
B_PAD = NW * BPW
TILE = 8
NT = BPW // TILE
RING = 4
NT_A = 284
NT_B = 108
PAD_EXTRA = (NT_A - NT_B) * TILE
C = 2 * D
CW = C // 2

_mesh = plsc.VectorSubcoreMesh(core_axis_name="c", subcore_axis_name="s")


def _tree_sum(vals):
    while len(vals) > 1:
        nxt = [vals[i] + vals[i + 1] for i in range(0, len(vals) - 1, 2)]
        if len(vals) % 2:
            nxt.append(vals[-1])
        vals = nxt
    return vals[0]


def _pack_bf16_words(a, b):
    ai = lax.bitcast_convert_type(a, jnp.int32)
    bi = lax.bitcast_convert_type(b, jnp.int32)
    ar = ai + jnp.int32(0x7FFF) + ((ai >> 16) & jnp.int32(1))
    br = bi + jnp.int32(0x7FFF) + ((bi >> 16) & jnp.int32(1))
    lo = (ar >> 16) & jnp.int32(0xFFFF)
    hi = br & jnp.int32(-0x10000)
    return hi | lo


def _accum(selfb, rows, stage):
    def node(t, c):
        r0 = t * S
        for p in range(D // 32):
            lo = pl.ds(32 * p, 16)
            hi = pl.ds(32 * p + 16, 16)
            stage[t, pl.ds(16 * p, 16)] = _pack_bf16_words(
                selfb[t, lo], selfb[t, hi])
            sa = _tree_sum([rows[r0 + j, lo] for j in range(S)])
            sb = _tree_sum([rows[r0 + j, hi] for j in range(S)])
            stage[t, pl.ds(D // 2 + 16 * p, 16)] = _pack_bf16_words(sa, sb)
        return c

    lax.fori_loop(0, TILE, node, 0)


@functools.partial(
    pl.kernel,
    out_type=jax.ShapeDtypeStruct((B_PAD, CW), jnp.int32),
    mesh=_mesh,
    scratch_types=(
        (pltpu.VMEM((NT_A * TILE,), jnp.int32),
         pltpu.VMEM((NT_A * TILE * S,), jnp.int32))
        + tuple(pltpu.VMEM((TILE, D), jnp.float32)
                for _ in range(RING))
        + tuple(pltpu.VMEM((TILE * S, D), jnp.float32)
                for _ in range(RING))
        + tuple(pltpu.VMEM((TILE, CW), jnp.int32)
                for _ in range(RING))
        + tuple(pltpu.SemaphoreType.DMA for _ in range(3 * RING))
    ),
)
def _sc_gather(nodes_hbm, neigh_hbm, feat_hbm, comb_out,
               idx_sv, idx_nv, *bufs):
    cid = lax.axis_index("c")
    sid = lax.axis_index("s")
    nt = lax.select(cid == 0, NT_A, NT_B)
    base_tile = lax.select(cid == 0, sid * NT_A, 16 * NT_A + sid * NT_B)
    base = base_tile * TILE
    selfbs = bufs[:RING]
    rows = bufs[RING:2 * RING]
    stages = bufs[2 * RING:3 * RING]
    ssem = bufs[3 * RING:4 * RING]
    gsem = bufs[4 * RING:5 * RING]
    wsem = bufs[5 * RING:6 * RING]

    pltpu.sync_copy(nodes_hbm.at[pl.ds(base, NT_A * TILE)], idx_sv)
    pltpu.sync_copy(neigh_hbm.at[pl.ds(base * S, NT_A * TILE * S)], idx_nv)

    def _sidx(i):
        return idx_sv.at[pl.ds(i * TILE, TILE)]

    def _nidx(i):
        return idx_nv.at[pl.ds(i * TILE * S, TILE * S)]

    for b in range(RING):
        pltpu.async_copy(feat_hbm.at[_sidx(b)], selfbs[b], ssem[b])
        pltpu.async_copy(feat_hbm.at[_nidx(b)], rows[b], gsem[b])

    def body(k, c):
        for b in range(RING):
            i = k * RING + b
            st = stages[b]
            pltpu.make_async_copy(feat_hbm.at[_sidx(i)], selfbs[b],
                                  ssem[b]).wait()
            pltpu.make_async_copy(feat_hbm.at[_nidx(i)], rows[b],
                                  gsem[b]).wait()

            def _wait_prev_write():
                pltpu.make_async_copy(
                    st,
                    comb_out.at[pl.ds(base + (i - RING) * TILE, TILE)],
                    wsem[b]).wait()

            pl.when(k > 0)(_wait_prev_write)

            _accum(selfbs[b], rows[b], st)
            pltpu.async_copy(
                st, comb_out.at[pl.ds(base + i * TILE, TILE)], wsem[b])

            def _next_gather():
                pltpu.async_copy(feat_hbm.at[_sidx(i + RING)],
                                 selfbs[b], ssem[b])
                pltpu.async_copy(feat_hbm.at[_nidx(i + RING)],
                                 rows[b], gsem[b])

            pl.when(k < nt // RING - 1)(_next_gather)
        return c

    lax.fori_loop(0, nt // RING, body, 0)
    for b in range(RING):
        pltpu.make_async_copy(
            stages[b],
            comb_out.at[pl.ds(base + (nt - RING + b) * TILE, TILE)],
            wsem[b]).wait()


_PERM = np.empty((C,), dtype=np.int32)
for _w in range(CW):
    _half = _w // 64
    _g = (_w % 64) // 16
    _k = _w % 16
    _PERM[_w] = 128 * _half + 32 * _g + _k
    _PERM[CW + _w] = 128 * _half + 32 * _g + 16 + _k

BLK = 1024
_DN = (((1,), (1,)), ((), ()))


def _tc_body(w_ref, comb_ref, out_ref):
    words = comb_ref[...]
    lo = lax.bitcast_convert_type(words << 16, jnp.float32)
    hi = lax.bitcast_convert_type(words & jnp.int32(-0x10000), jnp.float32)
    x = jnp.concatenate([lo, hi], axis=1).astype(jnp.bfloat16)
    out = lax.dot_general(w_ref[...].astype(jnp.bfloat16), x,
                          _DN, preferred_element_type=jnp.float32)
    out_ref[...] = jnp.maximum(out, 0.0)


def _tc_call(b):
    return pl.pallas_call(
        _tc_body,
        grid=(B_PAD // BLK,),
        in_specs=[
            pl.BlockSpec((EMBED, C), lambda i: (0, 0)),
            pl.BlockSpec((BLK, CW), lambda i: (i, 0)),
        ],
        out_specs=pl.BlockSpec((EMBED, BLK), lambda i: (0, i)),
        out_shape=jax.ShapeDtypeStruct((EMBED, b), jnp.float32),
    )


def kernel(nodes, neigh_idx, features, weight, pos_index, neg_index):
    del pos_index, neg_index
    b = nodes.shape[0]
    nodes_p = jnp.pad(nodes.astype(jnp.int32), (0, B_PAD - b + PAD_EXTRA))
    neigh_p = jnp.pad(neigh_idx.astype(jnp.int32).reshape(-1),
                      (0, (B_PAD - b + PAD_EXTRA) * S))
    comb_words = _sc_gather(nodes_p, neigh_p, features)
    w2 = jnp.concatenate(
        [weight[:, :D], weight[:, D:] * jnp.float32(1.0 / S)], axis=1)
    w2 = w2[:, _PERM]
    return _tc_call(b)(w2, comb_words)


# speedup vs baseline: 2.0113x; 1.0550x over previous
import functools

import numpy as np

import jax
import jax.numpy as jnp
from jax import lax
from jax.experimental import pallas as pl
from jax.experimental.pallas import tpu as pltpu
from jax.experimental.pallas import tpu_sc as plsc

D = 128
EMBED = 128
S = 10
NW = 32
BPW = 1568
B_PAD = NW * BPW
TILE = 8
NT = BPW // TILE
RING = 4
NT_A = 284
NT_B = 108
PAD_EXTRA = (NT_A - NT_B) * TILE
C = 2 * D
CW = C // 2

_mesh = plsc.VectorSubcoreMesh(core_axis_name="c", subcore_axis_name="s")


def _tree_sum(vals):
    while len(vals) > 1:
        nxt = [vals[i] + vals[i + 1] for i in range(0, len(vals) - 1, 2)]
        if len(vals) % 2:
            nxt.append(vals[-1])
        vals = nxt
    return vals[0]


def _pack_bf16_words(a, b):
    ai = lax.bitcast_convert_type(a, jnp.int32)
    bi = lax.bitcast_convert_type(b, jnp.int32)
    ar = ai + jnp.int32(0x7FFF) + ((ai >> 16) & jnp.int32(1))
    br = bi + jnp.int32(0x7FFF) + ((bi >> 16) & jnp.int32(1))
    lo = (ar >> 16) & jnp.int32(0xFFFF)
    hi = br & jnp.int32(-0x10000)
    return hi | lo


def _accum(selfb, rows, stage):
    def node(t, c):
        r0 = t * S
        for p in range(D // 32):
            lo = pl.ds(32 * p, 16)
            hi = pl.ds(32 * p + 16, 16)
            stage[t, pl.ds(16 * p, 16)] = _pack_bf16_words(
                selfb[t, lo], selfb[t, hi])
            sa = _tree_sum([rows[r0 + j, lo] for j in range(S)])
            sb = _tree_sum([rows[r0 + j, hi] for j in range(S)])
            stage[t, pl.ds(D // 2 + 16 * p, 16)] = _pack_bf16_words(sa, sb)
        return c

    lax.fori_loop(0, TILE, node, 0)


@functools.partial(
    pl.kernel,
    out_type=jax.ShapeDtypeStruct((B_PAD, CW), jnp.int32),
    mesh=_mesh,
    scratch_types=(
        (pltpu.VMEM((NT_A * TILE,), jnp.int32),
         pltpu.VMEM((NT_A * TILE * S,), jnp.int32))
        + tuple(pltpu.VMEM((TILE, D), jnp.float32)
                for _ in range(RING))
        + tuple(pltpu.VMEM((TILE * S, D), jnp.float32)
                for _ in range(RING))
        + tuple(pltpu.VMEM((TILE, CW), jnp.int32)
                for _ in range(RING))
        + tuple(pltpu.SemaphoreType.DMA for _ in range(3 * RING))
    ),
)
def _sc_gather(nodes_hbm, neigh_hbm, feat_hbm, comb_out,
               idx_sv, idx_nv, *bufs):
    cid = lax.axis_index("c")
    sid = lax.axis_index("s")
    nt = lax.select(cid == 0, NT_A, NT_B)
    base_tile = lax.select(cid == 0, sid * NT_A, 16 * NT_A + sid * NT_B)
    base = base_tile * TILE
    selfbs = bufs[:RING]
    rows = bufs[RING:2 * RING]
    stages = bufs[2 * RING:3 * RING]
    ssem = bufs[3 * RING:4 * RING]
    gsem = bufs[4 * RING:5 * RING]
    wsem = bufs[5 * RING:6 * RING]

    pltpu.sync_copy(nodes_hbm.at[pl.ds(base, NT_A * TILE)], idx_sv)
    pltpu.sync_copy(neigh_hbm.at[pl.ds(base * S, NT_A * TILE * S)], idx_nv)

    def _sidx(i):
        return idx_sv.at[pl.ds(i * TILE, TILE)]

    def _nidx(i):
        return idx_nv.at[pl.ds(i * TILE * S, TILE * S)]

    for b in range(RING):
        pltpu.async_copy(feat_hbm.at[_sidx(b)], selfbs[b], ssem[b])
        pltpu.async_copy(feat_hbm.at[_nidx(b)], rows[b], gsem[b])

    def body(k, c):
        for b in range(RING):
            i = k * RING + b
            st = stages[b]
            pltpu.make_async_copy(feat_hbm.at[_sidx(i)], selfbs[b],
                                  ssem[b]).wait()
            pltpu.make_async_copy(feat_hbm.at[_nidx(i)], rows[b],
                                  gsem[b]).wait()

            def _wait_prev_write():
                pltpu.make_async_copy(
                    st,
                    comb_out.at[pl.ds(base + (i - RING) * TILE, TILE)],
                    wsem[b]).wait()

            pl.when(k > 0)(_wait_prev_write)

            _accum(selfbs[b], rows[b], st)
            pltpu.async_copy(
                st, comb_out.at[pl.ds(base + i * TILE, TILE)], wsem[b])

            def _next_gather():
                pltpu.async_copy(feat_hbm.at[_sidx(i + RING)],
                                 selfbs[b], ssem[b])
                pltpu.async_copy(feat_hbm.at[_nidx(i + RING)],
                                 rows[b], gsem[b])

            pl.when(k < nt // RING - 1)(_next_gather)
        return c

    lax.fori_loop(0, nt // RING, body, 0)
    for b in range(RING):
        pltpu.make_async_copy(
            stages[b],
            comb_out.at[pl.ds(base + (nt - RING + b) * TILE, TILE)],
            wsem[b]).wait()


_PERM = np.empty((C,), dtype=np.int32)
for _w in range(CW):
    _half = _w // 64
    _g = (_w % 64) // 16
    _k = _w % 16
    _PERM[_w] = 128 * _half + 32 * _g + _k
    _PERM[CW + _w] = 128 * _half + 32 * _g + 16 + _k

BLK = 3584
_DN = (((1,), (1,)), ((), ()))


def _tc_body(w_ref, comb_ref, out_ref):
    words = comb_ref[...]
    lo = lax.bitcast_convert_type(words << 16, jnp.float32)
    hi = lax.bitcast_convert_type(words & jnp.int32(-0x10000), jnp.float32)
    x = jnp.concatenate([lo, hi], axis=1).astype(jnp.bfloat16)
    out = lax.dot_general(w_ref[...].astype(jnp.bfloat16), x,
                          _DN, preferred_element_type=jnp.float32)
    out_ref[...] = jnp.maximum(out, 0.0)


def _tc_call(b):
    return pl.pallas_call(
        _tc_body,
        grid=(B_PAD // BLK,),
        in_specs=[
            pl.BlockSpec((EMBED, C), lambda i: (0, 0)),
            pl.BlockSpec((BLK, CW), lambda i: (i, 0)),
        ],
        out_specs=pl.BlockSpec((EMBED, BLK), lambda i: (0, i)),
        out_shape=jax.ShapeDtypeStruct((EMBED, b), jnp.float32),
    )


def kernel(nodes, neigh_idx, features, weight, pos_index, neg_index):
    del pos_index, neg_index
    b = nodes.shape[0]
    nodes_p = jnp.pad(nodes.astype(jnp.int32), (0, B_PAD - b + PAD_EXTRA))
    neigh_p = jnp.pad(neigh_idx.astype(jnp.int32).reshape(-1),
                      (0, (B_PAD - b + PAD_EXTRA) * S))
    comb_words = _sc_gather(nodes_p, neigh_p, features)
    w2 = jnp.concatenate(
        [weight[:, :D], weight[:, D:] * jnp.float32(1.0 / S)], axis=1)
    w2 = w2[:, _PERM]
    return _tc_call(b)(w2, comb_words)


# speedup vs baseline: 2.0313x; 1.0099x over previous
import functools

import numpy as np

import jax
import jax.numpy as jnp
from jax import lax
from jax.experimental import pallas as pl
from jax.experimental.pallas import tpu as pltpu
from jax.experimental.pallas import tpu_sc as plsc

D = 128
EMBED = 128
S = 10
NW = 32
BPW = 1568
B_PAD = NW * BPW
TILE = 8
NT = BPW // TILE
RING = 4
NT_A = 284
NT_B = 108
PAD_EXTRA = (NT_A - NT_B) * TILE
C = 2 * D
CW = C // 2

_mesh = plsc.VectorSubcoreMesh(core_axis_name="c", subcore_axis_name="s")


def _tree_sum(vals):
    while len(vals) > 1:
        nxt = [vals[i] + vals[i + 1] for i in range(0, len(vals) - 1, 2)]
        if len(vals) % 2:
            nxt.append(vals[-1])
        vals = nxt
    return vals[0]


def _pack_bf16_words(a, b):
    ai = lax.bitcast_convert_type(a, jnp.int32)
    bi = lax.bitcast_convert_type(b, jnp.int32)
    ar = ai + jnp.int32(0x7FFF) + ((ai >> 16) & jnp.int32(1))
    br = bi + jnp.int32(0x7FFF) + ((bi >> 16) & jnp.int32(1))
    lo = (ar >> 16) & jnp.int32(0xFFFF)
    hi = br & jnp.int32(-0x10000)
    return hi | lo


def _accum(selfb, rows, stage):
    def node(t, c):
        r0 = t * S
        for p in range(D // 32):
            lo = pl.ds(32 * p, 16)
            hi = pl.ds(32 * p + 16, 16)
            stage[t, pl.ds(16 * p, 16)] = _pack_bf16_words(
                selfb[t, lo], selfb[t, hi])
            sa = _tree_sum([rows[r0 + j, lo] for j in range(S)])
            sb = _tree_sum([rows[r0 + j, hi] for j in range(S)])
            stage[t, pl.ds(D // 2 + 16 * p, 16)] = _pack_bf16_words(sa, sb)
        return c

    lax.fori_loop(0, TILE, node, 0)


@functools.partial(
    pl.kernel,
    out_type=jax.ShapeDtypeStruct((B_PAD, CW), jnp.int32),
    mesh=_mesh,
    scratch_types=(
        (pltpu.VMEM((NT_A * TILE,), jnp.int32),
         pltpu.VMEM((NT_A * TILE * S,), jnp.int32))
        + tuple(pltpu.VMEM((TILE, D), jnp.float32)
                for _ in range(RING))
        + tuple(pltpu.VMEM((TILE * S, D), jnp.float32)
                for _ in range(RING))
        + tuple(pltpu.VMEM((TILE, CW), jnp.int32)
                for _ in range(RING))
        + tuple(pltpu.SemaphoreType.DMA for _ in range(3 * RING))
    ),
)
def _sc_gather(nodes_hbm, neigh_hbm, feat_hbm, comb_out,
               idx_sv, idx_nv, *bufs):
    cid = lax.axis_index("c")
    sid = lax.axis_index("s")
    nt = lax.select(cid == 0, NT_A, NT_B)
    base_tile = lax.select(cid == 0, sid * NT_A, 16 * NT_A + sid * NT_B)
    base = base_tile * TILE
    selfbs = bufs[:RING]
    rows = bufs[RING:2 * RING]
    stages = bufs[2 * RING:3 * RING]
    ssem = bufs[3 * RING:4 * RING]
    gsem = bufs[4 * RING:5 * RING]
    wsem = bufs[5 * RING:6 * RING]

    pltpu.sync_copy(nodes_hbm.at[pl.ds(base, NT_A * TILE)], idx_sv)
    pltpu.sync_copy(neigh_hbm.at[pl.ds(base * S, NT_A * TILE * S)], idx_nv)

    def _sidx(i):
        return idx_sv.at[pl.ds(i * TILE, TILE)]

    def _nidx(i):
        return idx_nv.at[pl.ds(i * TILE * S, TILE * S)]

    for b in range(RING):
        pltpu.async_copy(feat_hbm.at[_sidx(b)], selfbs[b], ssem[b])
        pltpu.async_copy(feat_hbm.at[_nidx(b)], rows[b], gsem[b])

    def body(k, c):
        for b in range(RING):
            i = k * RING + b
            st = stages[b]
            pltpu.make_async_copy(feat_hbm.at[_sidx(i)], selfbs[b],
                                  ssem[b]).wait()
            pltpu.make_async_copy(feat_hbm.at[_nidx(i)], rows[b],
                                  gsem[b]).wait()

            def _wait_prev_write():
                pltpu.make_async_copy(
                    st,
                    comb_out.at[pl.ds(base + (i - RING) * TILE, TILE)],
                    wsem[b]).wait()

            pl.when(k > 0)(_wait_prev_write)

            _accum(selfbs[b], rows[b], st)
            pltpu.async_copy(
                st, comb_out.at[pl.ds(base + i * TILE, TILE)], wsem[b])

            def _next_gather():
                pltpu.async_copy(feat_hbm.at[_sidx(i + RING)],
                                 selfbs[b], ssem[b])
                pltpu.async_copy(feat_hbm.at[_nidx(i + RING)],
                                 rows[b], gsem[b])

            pl.when(k < nt // RING - 1)(_next_gather)
        return c

    lax.fori_loop(0, nt // RING, body, 0)
    for b in range(RING):
        pltpu.make_async_copy(
            stages[b],
            comb_out.at[pl.ds(base + (nt - RING + b) * TILE, TILE)],
            wsem[b]).wait()


_PERM = np.empty((C,), dtype=np.int32)
for _w in range(CW):
    _half = _w // 64
    _g = (_w % 64) // 16
    _k = _w % 16
    _PERM[_w] = 128 * _half + 32 * _g + _k
    _PERM[CW + _w] = 128 * _half + 32 * _g + 16 + _k

BLK = 7168
_DN = (((1,), (1,)), ((), ()))


def _tc_body(w_ref, comb_ref, out_ref):
    words = comb_ref[...]
    lo = lax.bitcast_convert_type(words << 16, jnp.float32)
    hi = lax.bitcast_convert_type(words & jnp.int32(-0x10000), jnp.float32)
    x = jnp.concatenate([lo, hi], axis=1).astype(jnp.bfloat16)
    out = lax.dot_general(w_ref[...].astype(jnp.bfloat16), x,
                          _DN, preferred_element_type=jnp.float32)
    out_ref[...] = jnp.maximum(out, 0.0)


def _tc_call(b):
    return pl.pallas_call(
        _tc_body,
        grid=(B_PAD // BLK,),
        in_specs=[
            pl.BlockSpec((EMBED, C), lambda i: (0, 0)),
            pl.BlockSpec((BLK, CW), lambda i: (i, 0)),
        ],
        out_specs=pl.BlockSpec((EMBED, BLK), lambda i: (0, i)),
        out_shape=jax.ShapeDtypeStruct((EMBED, b), jnp.float32),
    )


def kernel(nodes, neigh_idx, features, weight, pos_index, neg_index):
    del pos_index, neg_index
    b = nodes.shape[0]
    nodes_p = jnp.pad(nodes.astype(jnp.int32), (0, B_PAD - b + PAD_EXTRA))
    neigh_p = jnp.pad(neigh_idx.astype(jnp.int32).reshape(-1),
                      (0, (B_PAD - b + PAD_EXTRA) * S))
    comb_words = _sc_gather(nodes_p, neigh_p, features)
    w2 = jnp.concatenate(
        [weight[:, :D], weight[:, D:] * jnp.float32(1.0 / S)], axis=1)
    w2 = w2[:, _PERM]
    return _tc_call(b)(w2, comb_words)
